# baseline (speedup 1.0000x reference)
import jax
import jax.numpy as jnp
from jax.experimental import pallas as pl
from jax.experimental.pallas import tpu as pltpu

_N = 10000
_INFEAT = 128
_HIDDEN = 24
_OUTFEAT = 128
_BM = 400


_BH = _BM // 2


def _body(x_ref, adj_a_ref, adj_b_ref, w1_ref, b1_ref, w2_ref, b2_ref,
          o_ref, s_ref):
    @pl.when(pl.program_id(0) == 0)
    def _():
        s_ref[...] = jnp.dot(x_ref[...], w1_ref[...],
                             preferred_element_type=jnp.float32)

    acc_a = jnp.dot(adj_a_ref[...], s_ref[...],
                    preferred_element_type=jnp.float32)
    acc_b = jnp.dot(adj_b_ref[...], s_ref[...],
                    preferred_element_type=jnp.float32)
    h = jnp.concatenate([acc_a, acc_b], axis=0) + b1_ref[...]
    h = jnp.where(h > 0, h, 0.01 * h)
    o_ref[...] = jnp.tanh(
        jnp.dot(h, w2_ref[...], preferred_element_type=jnp.float32)
        + b2_ref[...])


def kernel(x, adj, W1, b1, W2, b2):
    b1r = b1.reshape(1, _HIDDEN)
    b2r = b2.reshape(1, _OUTFEAT)

    return pl.pallas_call(
        _body,
        grid=(_N // _BM,),
        in_specs=[
            pl.BlockSpec((_N, _INFEAT), lambda i: (0, 0)),
            pl.BlockSpec((_BH, _N), lambda i: (2 * i, 0)),
            pl.BlockSpec((_BH, _N), lambda i: (2 * i + 1, 0)),
            pl.BlockSpec((_INFEAT, _HIDDEN), lambda i: (0, 0)),
            pl.BlockSpec((1, _HIDDEN), lambda i: (0, 0)),
            pl.BlockSpec((_HIDDEN, _OUTFEAT), lambda i: (0, 0)),
            pl.BlockSpec((1, _OUTFEAT), lambda i: (0, 0)),
        ],
        out_specs=pl.BlockSpec((_BM, _OUTFEAT), lambda i: (i, 0)),
        out_shape=jax.ShapeDtypeStruct((_N, _OUTFEAT), jnp.float32),
        scratch_shapes=[pltpu.VMEM((_N, _HIDDEN), jnp.float32)],
    )(x, adj, adj, W1, b1r, W2, b2r)
